# Initial kernel scaffold; baseline (speedup 1.0000x reference)
import jax
import jax.numpy as jnp
from jax.experimental import pallas as pl


def kernel(spatial_info, entity_embeddings, locations, W_proj, b_proj):
    raise NotImplementedError("write your pallas kernel here")


# speedup vs baseline: 5.8623x; 5.8623x over previous
import jax
import jax.numpy as jnp
from jax import lax
from jax.experimental import pallas as pl
from jax.experimental.pallas import tpu as pltpu


def _body(sp_ref, emb_ref, w_ref, b_ref, fidx_ref, out_ref, proj_ref, scat_ref):
    N = emb_ref.shape[1]
    C = sp_ref.shape[1]
    emb = emb_ref[0]
    proj = lax.dot_general(
        emb, w_ref[...], (((1,), (0,)), ((), ())),
        preferred_element_type=jnp.float32,
        precision=lax.Precision.HIGHEST,
    ) + b_ref[...]
    proj_ref[...] = jnp.maximum(proj, 0.0)
    scat_ref[...] = jnp.zeros_like(scat_ref)

    def step(n, carry):
        i = fidx_ref[0, 0, n]
        scat_ref[pl.ds(i, 1), :] = proj_ref[pl.ds(n, 1), :]
        return carry

    lax.fori_loop(0, N, step, 0)

    out_ref[0, :C, :] = sp_ref[0]
    out_ref[0, C:, :] = jnp.swapaxes(scat_ref[...], 0, 1)


def kernel(spatial_info, entity_embeddings, locations, W_proj, b_proj):
    B, C, H, W = spatial_info.shape
    N = entity_embeddings.shape[1]
    D = W_proj.shape[1]
    HW = H * W

    lh = jnp.clip(locations[..., 0], 0, H - 1)
    lw = jnp.clip(locations[..., 1], 0, W - 1)
    fidx = (lh * W + lw).astype(jnp.int32).reshape(B, 1, N)
    sp = spatial_info.reshape(B, C, HW)
    b2 = b_proj.reshape(1, D)

    out_flat = pl.pallas_call(
        _body,
        grid=(B,),
        in_specs=[
            pl.BlockSpec((1, C, HW), lambda b: (b, 0, 0)),
            pl.BlockSpec((1, N, entity_embeddings.shape[2]), lambda b: (b, 0, 0)),
            pl.BlockSpec(W_proj.shape, lambda b: (0, 0)),
            pl.BlockSpec((1, D), lambda b: (0, 0)),
            pl.BlockSpec((1, 1, N), lambda b: (b, 0, 0), memory_space=pltpu.SMEM),
        ],
        out_specs=pl.BlockSpec((1, C + D, HW), lambda b: (b, 0, 0)),
        out_shape=jax.ShapeDtypeStruct((B, C + D, HW), jnp.float32),
        scratch_shapes=[
            pltpu.VMEM((N, D), jnp.float32),
            pltpu.VMEM((HW, D), jnp.float32),
        ],
    )(sp, entity_embeddings, W_proj, b2, fidx)
    return out_flat.reshape(B, C + D, H, W)
